# baseline (speedup 1.0000x reference)
import functools

import jax
import jax.numpy as jnp
from jax.experimental import pallas as pl
from jax.experimental.pallas import tpu as pltpu

N = 4096
K = N // 2
LOGN = 12
ROWS = 128


def _row_step(xp, logd):
    n = xp.shape[0]
    m = 1 << (logd + 3)
    xr = xp.reshape(n // (2 * m), 2, m, ROWS)
    mx = jnp.maximum(xr[:, 0], xr[:, 1])
    mn = jnp.minimum(xr[:, 0], xr[:, 1])
    return jnp.stack([mx, mn], axis=1).reshape(n, ROWS)


def _sub_step(xp, q, logd):
    dp = 1 << (logd - 9)
    pu = jnp.roll(xp, dp, axis=0)
    pd = jnp.roll(xp, -dp, axis=0)
    lower = (q & dp) == 0
    return jnp.where(lower, jnp.maximum(xp, pd), jnp.minimum(xp, pu))


def _plane_pair_step(ys, logd):
    hp = 1 << (logd - 9)
    out = list(ys)
    for h in range(8):
        if h & hp:
            continue
        h2 = h | hp
        out[h] = jnp.maximum(ys[h], ys[h2])
        out[h2] = jnp.minimum(ys[h], ys[h2])
    return out


def _sort_body(x_ref, o_ref):
    x = x_ref[...]
    parts = [jnp.transpose(x[:, h * 512:(h + 1) * 512]) for h in range(8)]
    xp = jnp.stack(parts, axis=1).reshape(N, ROWS)

    q = jax.lax.broadcasted_iota(jnp.int32, (N, 1), 0)
    j = (q % 8) * 512 + (q // 8)

    def sgn(k):
        return jnp.where((j & k) == 0, jnp.float32(1), jnp.float32(-1))

    xp = xp * sgn(2)
    for logk in range(1, 10):
        for logd in range(logk - 1, -1, -1):
            xp = _row_step(xp, logd)
        xp = xp * (sgn(1 << logk) * sgn(2 << logk))

    ys = [xp.reshape(N // 8, 8, ROWS)[:, h, :] for h in range(8)]
    ys = _plane_pair_step(ys, 9)
    xp = jnp.stack(ys, axis=1).reshape(N, ROWS)
    for logd in range(8, -1, -1):
        xp = _row_step(xp, logd)
    xp = xp * (sgn(1 << 10) * sgn(1 << 11))

    ys = [xp.reshape(N // 8, 8, ROWS)[:, h, :] for h in range(8)]
    for logd in (10, 9):
        ys = _plane_pair_step(ys, logd)
    xp = jnp.stack(ys, axis=1).reshape(N, ROWS)
    for logd in range(8, -1, -1):
        xp = _row_step(xp, logd)
    xp = xp * sgn(1 << 11)

    ys = [xp.reshape(N // 8, 8, ROWS)[:, h, :] for h in range(8)]
    for logd in (11, 10, 9):
        ys = _plane_pair_step(ys, logd)
    ys = [_row_step(y, 5) for y in ys[:4]]

    semi = [ys[s & 3].reshape(2, 256, ROWS)[s >> 2] for s in range(8)]
    z = jnp.stack(semi, axis=1).reshape(K, ROWS)
    for logd in range(7, -1, -1):
        z = _row_step(z, logd)

    zs = z.reshape(K // 8, 8, ROWS)
    for s in range(8):
        base = (s & 1) * 512 + ((s >> 1) & 1) * 1024 + (s >> 2) * 256
        o_ref[:, base:base + 256] = jnp.transpose(zs[:, s, :])


@jax.jit
def kernel(x):
    b, t, n = x.shape
    rows = b * t
    flat = x.reshape(rows, n)
    out = pl.pallas_call(
        _sort_body,
        grid=(rows // ROWS,),
        in_specs=[pl.BlockSpec((ROWS, N), lambda i: (i, 0))],
        out_specs=pl.BlockSpec((ROWS, K), lambda i: (i, 0)),
        out_shape=jax.ShapeDtypeStruct((rows, K), jnp.float32),
        compiler_params=pltpu.CompilerParams(
            dimension_semantics=("arbitrary",),
        ),
    )(flat)
    return out.reshape(b, t, K)


# speedup vs baseline: 1.0590x; 1.0590x over previous
import functools

import jax
import jax.numpy as jnp
from jax.experimental import pallas as pl
from jax.experimental.pallas import tpu as pltpu

N = 4096
K = N // 2
LOGN = 12
ROWS = 128


def _row_step(xp, logd):
    n = xp.shape[0]
    m = 1 << (logd + 3)
    xr = xp.reshape(n // (2 * m), 2, m, ROWS)
    mx = jnp.maximum(xr[:, 0], xr[:, 1])
    mn = jnp.minimum(xr[:, 0], xr[:, 1])
    return jnp.stack([mx, mn], axis=1).reshape(n, ROWS)


def _row_step2(xp, logd):
    n = xp.shape[0]
    mb = 1 << (logd + 2)
    xr = xp.reshape(n // (4 * mb), 2, 2, mb, ROWS)
    a, b = xr[:, 0, 0], xr[:, 0, 1]
    c, d = xr[:, 1, 0], xr[:, 1, 1]
    hi0, lo0 = jnp.maximum(a, c), jnp.minimum(a, c)
    hi1, lo1 = jnp.maximum(b, d), jnp.minimum(b, d)
    o0, o1 = jnp.maximum(hi0, hi1), jnp.minimum(hi0, hi1)
    o2, o3 = jnp.maximum(lo0, lo1), jnp.minimum(lo0, lo1)
    return jnp.stack([o0, o1, o2, o3], axis=1).reshape(n, ROWS)


def _row_run(xp, logds):
    i = 0
    while i < len(logds):
        if i + 1 < len(logds):
            xp = _row_step2(xp, logds[i])
            i += 2
        else:
            xp = _row_step(xp, logds[i])
            i += 1
    return xp


def _sub_step(xp, q, logd):
    dp = 1 << (logd - 9)
    pu = jnp.roll(xp, dp, axis=0)
    pd = jnp.roll(xp, -dp, axis=0)
    lower = (q & dp) == 0
    return jnp.where(lower, jnp.maximum(xp, pd), jnp.minimum(xp, pu))


def _plane_pair_step(ys, logd):
    hp = 1 << (logd - 9)
    out = list(ys)
    for h in range(8):
        if h & hp:
            continue
        h2 = h | hp
        out[h] = jnp.maximum(ys[h], ys[h2])
        out[h2] = jnp.minimum(ys[h], ys[h2])
    return out


def _sort_body(x_ref, o_ref):
    x = x_ref[...]
    parts = [jnp.transpose(x[:, h * 512:(h + 1) * 512]) for h in range(8)]
    xp = jnp.stack(parts, axis=1).reshape(N, ROWS)

    q = jax.lax.broadcasted_iota(jnp.int32, (N, 1), 0)
    j = (q % 8) * 512 + (q // 8)

    def sgn(k):
        return jnp.where((j & k) == 0, jnp.float32(1), jnp.float32(-1))

    xp = xp * sgn(2)
    for logk in range(1, 10):
        xp = _row_run(xp, list(range(logk - 1, -1, -1)))
        xp = xp * (sgn(1 << logk) * sgn(2 << logk))

    xp = _sub_step(xp, q, 9)
    xp = _row_run(xp, list(range(8, -1, -1)))
    xp = xp * (sgn(1 << 10) * sgn(1 << 11))

    ys = [xp.reshape(N // 8, 8, ROWS)[:, h, :] for h in range(8)]
    for logd in (10, 9):
        ys = _plane_pair_step(ys, logd)
    xp = jnp.stack(ys, axis=1).reshape(N, ROWS)
    xp = _row_run(xp, list(range(8, -1, -1)))
    xp = xp * sgn(1 << 11)

    ys = [xp.reshape(N // 8, 8, ROWS)[:, h, :] for h in range(8)]
    for logd in (11, 10, 9):
        ys = _plane_pair_step(ys, logd)
    ys = [_row_step(y, 5) for y in ys[:4]]

    semi = [ys[s & 3].reshape(2, 256, ROWS)[s >> 2] for s in range(8)]
    z = jnp.stack(semi, axis=1).reshape(K, ROWS)
    z = _row_run(z, list(range(7, -1, -1)))

    zs = z.reshape(K // 8, 8, ROWS)
    for s in range(8):
        base = (s & 1) * 512 + ((s >> 1) & 1) * 1024 + (s >> 2) * 256
        o_ref[:, base:base + 256] = jnp.transpose(zs[:, s, :])


@jax.jit
def kernel(x):
    b, t, n = x.shape
    rows = b * t
    flat = x.reshape(rows, n)
    out = pl.pallas_call(
        _sort_body,
        grid=(rows // ROWS,),
        in_specs=[pl.BlockSpec((ROWS, N), lambda i: (i, 0))],
        out_specs=pl.BlockSpec((ROWS, K), lambda i: (i, 0)),
        out_shape=jax.ShapeDtypeStruct((rows, K), jnp.float32),
        compiler_params=pltpu.CompilerParams(
            dimension_semantics=("arbitrary",),
        ),
    )(flat)
    return out.reshape(b, t, K)
